# baseline (speedup 1.0000x reference)
import dataclasses
import functools

import jax
import jax.numpy as jnp
from jax import lax
from jax.experimental import pallas as pl
from jax.experimental.pallas import tpu as pltpu
from jax.experimental.pallas import tpu_sc as plsc

_NUM_CLASSES = 1000
_LS = 0.1
_COLD = _LS / (_NUM_CLASSES - 1)
_HOT = (1.0 - _LS) + _COLD

_B = 1024
_S = 50
_SPAD = 64
_L = 16
_NW = 32
_BPW = _B // _NW


def kernel(x_i):
    xpad = jnp.pad(x_i, ((0, 0), (0, _SPAD - _S)))

    mesh = plsc.VectorSubcoreMesh(core_axis_name="c", subcore_axis_name="s")
    cp = pltpu.CompilerParams()
    if "needs_layout_passes" in pltpu.CompilerParams.__dataclass_fields__:
        cp = dataclasses.replace(cp, needs_layout_passes=False)

    @functools.partial(
        pl.kernel,
        out_type=jax.ShapeDtypeStruct((_B, _S, _NUM_CLASSES), jnp.float32),
        mesh=mesh,
        compiler_params=cp,
        scratch_types=[
            pltpu.VMEM((_BPW, _SPAD), jnp.int32),
            pltpu.VMEM((_S, _NUM_CLASSES), jnp.float32),
            pltpu.VMEM((_S, _NUM_CLASSES), jnp.float32),
            pltpu.SemaphoreType.DMA,
            pltpu.SemaphoreType.DMA,
            pltpu.SemaphoreType.DMA,
        ],
    )
    def sc_onehot(x_hbm, o_hbm, idx_v, buf_a, buf_b, sem_i, sem_a, sem_b):
        wid = lax.axis_index("s") * 2 + lax.axis_index("c")
        base = wid * _BPW
        pltpu.async_copy(x_hbm.at[pl.ds(base, _BPW)], idx_v, sem_i).wait()

        cold16 = jnp.full((_L,), _COLD, jnp.float32)
        hot16 = jnp.full((_L,), _HOT, jnp.float32)
        iota16 = lax.iota(jnp.int32, _L)

        for buf in (buf_a, buf_b):

            @pl.loop(0, _S)
            def _(i, buf=buf):
                @pl.loop(0, _NUM_CLASSES // _L)
                def _(j, i=i, buf=buf):
                    buf[i, pl.ds(pl.multiple_of(j * _L, _L), _L)] = cold16

            tail = _NUM_CLASSES - (_NUM_CLASSES % _L)
            for g in range(4):
                s16 = iota16 + g * _L
                rmask = s16 < _S
                for cc in range(_NUM_CLASSES % _L):
                    col16 = jnp.full((_L,), tail + cc, jnp.int32)
                    plsc.store_scatter(buf, [s16, col16], cold16, mask=rmask)

        def scat(buf, t, val16):
            for k in range(_SPAD // _L):
                col16 = idx_v[t, pl.ds(k * _L, _L)]
                s16 = iota16 + (k * _L)
                plsc.store_scatter(buf, [s16, col16], val16, mask=s16 < _S)

        npair = _BPW // 2

        @pl.loop(0, npair)
        def _(p):
            for j, buf, sem in ((0, buf_a, sem_a), (1, buf_b, sem_b)):
                t = p * 2 + j
                b = base + t

                @pl.when(p > 0)
                def _(buf=buf, sem=sem, t=t, b=b):
                    pltpu.make_async_copy(buf, o_hbm.at[b], sem).wait()
                    scat(buf, t - 2, cold16)

                scat(buf, t, hot16)
                pltpu.async_copy(buf, o_hbm.at[b], sem)

        pltpu.make_async_copy(buf_a, o_hbm.at[base], sem_a).wait()
        pltpu.make_async_copy(buf_b, o_hbm.at[base], sem_b).wait()

    return sc_onehot(xpad)


# speedup vs baseline: 1.0633x; 1.0633x over previous
import dataclasses
import functools

import jax
import jax.numpy as jnp
from jax import lax
from jax.experimental import pallas as pl
from jax.experimental.pallas import tpu as pltpu
from jax.experimental.pallas import tpu_sc as plsc

_NUM_CLASSES = 1000
_LS = 0.1
_COLD = _LS / (_NUM_CLASSES - 1)
_HOT = (1.0 - _LS) + _COLD

_B = 1024
_S = 50
_SPAD = 64
_L = 16
_NW = 32
_BPW = _B // _NW


def kernel(x_i):
    xpad = jnp.pad(x_i, ((0, 0), (0, _SPAD - _S)))

    mesh = plsc.VectorSubcoreMesh(core_axis_name="c", subcore_axis_name="s")
    cp = pltpu.CompilerParams()
    if "needs_layout_passes" in pltpu.CompilerParams.__dataclass_fields__:
        cp = dataclasses.replace(cp, needs_layout_passes=False)

    @functools.partial(
        pl.kernel,
        out_type=jax.ShapeDtypeStruct((_B, _S, _NUM_CLASSES), jnp.float32),
        mesh=mesh,
        compiler_params=cp,
        scratch_types=[
            pltpu.VMEM((_BPW, _SPAD), jnp.int32),
            pltpu.VMEM((_S, _NUM_CLASSES), jnp.float32),
            pltpu.VMEM((_S, _NUM_CLASSES), jnp.float32),
            pltpu.SemaphoreType.DMA,
            pltpu.SemaphoreType.DMA,
            pltpu.SemaphoreType.DMA,
        ],
    )
    def sc_onehot(x_hbm, o_hbm, idx_v, buf_a, buf_b, sem_i, sem_a, sem_b):
        wid = lax.axis_index("s") * 2 + lax.axis_index("c")
        base = wid * _BPW
        pltpu.async_copy(x_hbm.at[pl.ds(base, _BPW)], idx_v, sem_i).wait()

        cold16 = jnp.full((_L,), _COLD, jnp.float32)
        hot16 = jnp.full((_L,), _HOT, jnp.float32)
        iota16 = lax.iota(jnp.int32, _L)

        for buf in (buf_a, buf_b):

            @pl.loop(0, _S)
            def _(i, buf=buf):
                for j in range(_NUM_CLASSES // _L):
                    buf[i, pl.ds(pl.multiple_of(j * _L, _L), _L)] = cold16

            tail = _NUM_CLASSES - (_NUM_CLASSES % _L)
            for g in range(4):
                s16 = iota16 + g * _L
                rmask = s16 < _S
                for cc in range(_NUM_CLASSES % _L):
                    col16 = jnp.full((_L,), tail + cc, jnp.int32)
                    plsc.store_scatter(buf, [s16, col16], cold16, mask=rmask)

        def scat(buf, t, val16):
            for k in range(_SPAD // _L):
                col16 = idx_v[t, pl.ds(k * _L, _L)]
                s16 = iota16 + (k * _L)
                plsc.store_scatter(buf, [s16, col16], val16, mask=s16 < _S)

        npair = _BPW // 2

        @pl.loop(0, npair)
        def _(p):
            for j, buf, sem in ((0, buf_a, sem_a), (1, buf_b, sem_b)):
                t = p * 2 + j
                b = base + t

                @pl.when(p > 0)
                def _(buf=buf, sem=sem, t=t, b=b):
                    pltpu.make_async_copy(buf, o_hbm.at[b], sem).wait()
                    scat(buf, t - 2, cold16)

                scat(buf, t, hot16)
                pltpu.async_copy(buf, o_hbm.at[b], sem)

        pltpu.make_async_copy(buf_a, o_hbm.at[base], sem_a).wait()
        pltpu.make_async_copy(buf_b, o_hbm.at[base], sem_b).wait()

    return sc_onehot(xpad)


# speedup vs baseline: 5.3175x; 5.0010x over previous
import jax
import jax.numpy as jnp
from jax.experimental import pallas as pl

_NUM_CLASSES = 1000
_LS = 0.1
_COLD = _LS / (_NUM_CLASSES - 1)
_HOT = (1.0 - _LS) + _COLD

_C_BLK = 200
_B_BLK = 128


def _onehot_body(xt_ref, o_ref):
    ci = pl.program_id(0) * _C_BLK
    s, b = xt_ref.shape
    iota = ci + jax.lax.broadcasted_iota(jnp.int32, (s, _C_BLK, b), 1)
    o_ref[...] = jnp.where(
        xt_ref[...][:, None, :] == iota,
        jnp.float32(_HOT),
        jnp.float32(_COLD),
    )


def kernel(x_i):
    b, s = x_i.shape
    xt = x_i.T
    out_t = pl.pallas_call(
        _onehot_body,
        grid=(_NUM_CLASSES // _C_BLK, b // _B_BLK),
        in_specs=[pl.BlockSpec((s, _B_BLK), lambda i, j: (0, j))],
        out_specs=pl.BlockSpec((s, _C_BLK, _B_BLK), lambda i, j: (0, i, j)),
        out_shape=jax.ShapeDtypeStruct((s, _NUM_CLASSES, b), jnp.float32),
    )(xt)
    return out_t.transpose(2, 0, 1)
